# baseline (speedup 1.0000x reference)
import functools

import jax
import jax.numpy as jnp
from jax import lax
from jax.experimental import pallas as pl
from jax.experimental.pallas import tpu as pltpu
from jax.experimental.pallas import tpu_sc as plsc

B = 4096
F = 26
D = 64
LANES = 16
NUM_WORKERS = 32
ROWS_PER_W = B // NUM_WORKERS
CHUNK_ROWS = 32
NCHUNK = ROWS_PER_W // CHUNK_ROWS
NBUF = 2


def _emb_body(idx_hbm, table_hbm, out_hbm, idx_v, rows_v, out_v, sem0, sem1):
    sems = (sem0, sem1)
    cid = lax.axis_index("c")
    sid = lax.axis_index("s")
    wid = sid * 2 + cid
    obase = wid * ROWS_PER_W

    def load_idx(c, buf):
        col0 = obase + c * CHUNK_ROWS
        pltpu.sync_copy(
            idx_hbm.at[pl.ds(0, F), pl.ds(col0, CHUNK_ROWS)], idx_v.at[buf]
        )

    def start_gathers(buf):
        for f in range(F):
            pltpu.make_async_copy(
                table_hbm.at[idx_v.at[buf, f]], rows_v.at[buf, f], sems[buf]
            ).start()

    def wait_gathers(buf):
        for f in range(F):
            pltpu.make_async_copy(
                table_hbm.at[idx_v.at[buf, f]], rows_v.at[buf, f], sems[buf]
            ).wait()

    def compute_store(c, buf):
        def row_body(i, carry):
            for d in range(D // LANES):
                sl = pl.ds(d * LANES, LANES)
                acc = None
                for f in range(F):
                    v = rows_v[buf, f, i, sl]
                    acc = v if acc is None else acc + v
                out_v[i, sl] = acc
            return carry

        lax.fori_loop(0, CHUNK_ROWS, row_body, 0)
        orow = obase + c * CHUNK_ROWS
        pltpu.sync_copy(out_v, out_hbm.at[pl.ds(orow, CHUNK_ROWS)])

    load_idx(0, 0)
    start_gathers(0)

    def outer(it, carry):
        c2 = it * NBUF
        for b in range(NBUF):
            c = c2 + b
            nxt = c + 1

            @pl.when(nxt < NCHUNK)
            def _():
                load_idx(nxt, 1 - b)
                start_gathers(1 - b)

            wait_gathers(b)
            compute_store(c, b)
        return carry

    lax.fori_loop(0, NCHUNK // NBUF, outer, 0)


def kernel(x, table):
    xt = jnp.pad(x.T, ((0, 32 - F), (0, 0)))
    mesh = plsc.VectorSubcoreMesh(core_axis_name="c", subcore_axis_name="s")
    k = functools.partial(
        pl.kernel,
        mesh=mesh,
        out_type=jax.ShapeDtypeStruct((B, D), jnp.float32),
        scratch_types=[
            pltpu.VMEM((NBUF, F, CHUNK_ROWS), jnp.int32),
            pltpu.VMEM((NBUF, F, CHUNK_ROWS, D), jnp.float32),
            pltpu.VMEM((CHUNK_ROWS, D), jnp.float32),
            pltpu.SemaphoreType.DMA,
            pltpu.SemaphoreType.DMA,
        ],
        compiler_params=pltpu.CompilerParams(use_tc_tiling_on_sc=False),
    )(_emb_body)
    return k(xt, table)


# speedup vs baseline: 1.4480x; 1.4480x over previous
import functools

import jax
import jax.numpy as jnp
from jax import lax
from jax.experimental import pallas as pl
from jax.experimental.pallas import tpu as pltpu
from jax.experimental.pallas import tpu_sc as plsc

B = 4096
F = 26
D = 64
LANES = 16
FPAD = 32
NUM_WORKERS = 32
DIMS_PER_W = D // NUM_WORKERS
V = 100000
BC = 256
NCH = B // BC
NIB = 2


def _emb_body(idx_hbm, tab_hbm, out_hbm, row_v, idx_v, outc_v, sem0, sem1, semr):
    sems = (sem0, sem1)
    cid = lax.axis_index("c")
    sid = lax.axis_index("s")
    wid = sid * 2 + cid

    def load_idx(c, buf):
        pltpu.make_async_copy(
            idx_hbm.at[:, pl.ds(c * BC, BC)], idx_v.at[buf], sems[buf]
        ).start()

    def wait_idx(buf):
        pltpu.make_async_copy(
            idx_hbm.at[:, pl.ds(0, BC)], idx_v.at[buf], sems[buf]
        ).wait()

    def compute_chunk(d, c, buf):
        def jbody(j, carry):
            sl = pl.ds(j * LANES, LANES)
            acc = None
            for f in range(F):
                iv = idx_v[buf, f, sl]
                g = plsc.load_gather(row_v, [iv])
                acc = g if acc is None else acc + g
            outc_v[sl] = acc
            return carry

        lax.fori_loop(0, BC // LANES, jbody, 0)
        pltpu.sync_copy(outc_v, out_hbm.at[d, pl.ds(c * BC, BC)])

    for r in range(DIMS_PER_W):
        d = wid * DIMS_PER_W + r
        pltpu.make_async_copy(tab_hbm.at[d], row_v, semr).start()
        load_idx(0, 0)
        pltpu.make_async_copy(tab_hbm.at[d], row_v, semr).wait()

        def outer(it, carry):
            c2 = it * NIB
            for b in range(NIB):
                c = c2 + b

                @pl.when(c + 1 < NCH)
                def _():
                    load_idx(c + 1, 1 - b)

                wait_idx(b)
                compute_chunk(d, c, b)
            return carry

        lax.fori_loop(0, NCH // NIB, outer, 0)


def kernel(x, table):
    xt = jnp.pad(x.T, ((0, FPAD - F), (0, 0)))
    tt = table.T
    mesh = plsc.VectorSubcoreMesh(core_axis_name="c", subcore_axis_name="s")
    k = functools.partial(
        pl.kernel,
        mesh=mesh,
        out_type=jax.ShapeDtypeStruct((D, B), jnp.float32),
        scratch_types=[
            pltpu.VMEM((V,), jnp.float32),
            pltpu.VMEM((NIB, FPAD, BC), jnp.int32),
            pltpu.VMEM((BC,), jnp.float32),
            pltpu.SemaphoreType.DMA,
            pltpu.SemaphoreType.DMA,
            pltpu.SemaphoreType.DMA,
        ],
        compiler_params=pltpu.CompilerParams(
            use_tc_tiling_on_sc=True, needs_layout_passes=False
        ),
    )(_emb_body)
    out_t = k(xt, tt)
    return out_t.T
